# baseline (speedup 1.0000x reference)
import dataclasses
import functools

import jax
import jax.numpy as jnp
from jax import lax
from jax.experimental import pallas as pl
from jax.experimental.pallas import tpu as pltpu
from jax.experimental.pallas import tpu_sc as plsc

B = 16384
C = 1000
NTILE = 32
PER_TILE = B // NTILE
NBUCKET = 8
VECS = PER_TILE // 16
NCHUNK = PER_TILE // 128

_LN2 = 0.6931471805599453
_SQRT2 = 1.4142135


def _log16(x):
    bits = lax.bitcast_convert_type(x, jnp.int32)
    e = ((bits >> 23) & 0xFF) - 127
    m = lax.bitcast_convert_type((bits & 0x007FFFFF) | 0x3F800000, jnp.float32)
    big = m > _SQRT2
    m = jnp.where(big, m * 0.5, m)
    ef = (e + jnp.where(big, 1, 0)).astype(jnp.float32)
    s = (m - 1.0) / (m + 1.0)
    z = s * s
    p = 1.0 + z * (1 / 3 + z * (1 / 5 + z * (1 / 7 + z * (1 / 9))))
    return ef * _LN2 + 2.0 * s * p


def _sc_body(yp_hbm, yt_hbm, out_hbm, yt_v, rowbuf, lanebuf, ring0, ring1,
             b7buf, stage_v, sem0, sem1, sem7):
    wid = lax.axis_index("c") * 16 + lax.axis_index("s")
    base_row = wid * PER_TILE
    iota = lax.iota(jnp.int32, 16)
    dyn = wid >> 31

    pltpu.sync_copy(yt_hbm.at[pl.ds(base_row, PER_TILE)], yt_v)

    zeros16 = jnp.zeros((16,), jnp.int32)

    def mset(k, _):
        rowbuf[pl.ds(k * 16, 16)] = zeros16
        lanebuf[pl.ds(k * 16, 16)] = zeros16
        return 0

    lax.fori_loop(0, NBUCKET * PER_TILE // 16 + dyn, mset, 0)

    def build(v, bases):
        c = yt_v[pl.ds(v * 16, 16)]
        rows = base_row + v * 16 + iota
        b = c >> 7
        lane = c & 127
        new_bases = []
        for t in range(NBUCKET):
            m = b == t
            m_i = jnp.where(m, 1, 0)
            rank = plsc.cumsum(m_i) - 1
            pos = bases[t] + rank
            plsc.store_scatter(rowbuf, [pos], rows, mask=m)
            plsc.store_scatter(lanebuf, [pos], lane, mask=m)
            cnt = plsc.all_reduce_population_count(m)
            new_bases.append(bases[t] + cnt)
        return tuple(new_bases)

    bases0 = tuple(jnp.full((16,), t * PER_TILE, jnp.int32)
                   for t in range(NBUCKET))
    bases = lax.fori_loop(0, VECS + dyn, build, bases0)
    counts = [jnp.max(bases[t]) - t * PER_TILE for t in range(NBUCKET)]

    n7 = counts[7]

    def fire7(i, _):
        v = rowbuf[pl.ds(7 * PER_TILE + (i >> 4) * 16, 16)]
        row = jnp.sum(jnp.where(iota == (i & 15), v, 0))
        pltpu.async_copy(yp_hbm.at[row, pl.ds(896, 104)], b7buf.at[i], sem7)
        return 0

    lax.fori_loop(0, n7, fire7, 0)

    rings = (ring0, ring1)
    sems = (sem0, sem1)
    counts_vec = jnp.zeros((16,), jnp.int32)
    for t in range(NBUCKET - 1):
        counts_vec = jnp.where(iota == t, counts[t], counts_vec)

    def site_nt(t):
        return jnp.sum(jnp.where(iota == t, counts_vec, 0))

    def fire(k, p):
        t = k >> 2
        j = k & 3
        off = pl.multiple_of(t * PER_TILE + j * 128, 128)
        cb = pl.multiple_of(t * 128, 128)
        idx = rowbuf.at[pl.ds(off, 128)]
        pltpu.async_copy(yp_hbm.at[idx, pl.ds(cb, 128)], rings[p], sems[p])

    def drain_extract(k, p, acc):
        t = k >> 2
        j = k & 3
        off = pl.multiple_of(t * PER_TILE + j * 128, 128)
        cb = pl.multiple_of(t * 128, 128)
        idx = rowbuf.at[pl.ds(off, 128)]
        pltpu.make_async_copy(yp_hbm.at[idx, pl.ds(cb, 128)],
                              rings[p], sems[p]).wait()
        n_t = site_nt(t)
        dest = rings[p]

        def ext(h, a):
            lane16 = lanebuf[pl.ds(off + h * 16, 16)]
            slot16 = h * 16 + iota
            vals = plsc.load_gather(dest, [slot16, lane16])
            valid = (j * 128 + slot16) < n_t
            contrib = jnp.where(valid, _log16(vals + 1e-7), 0.0)
            return a + contrib

        hi = (jnp.minimum(128, n_t - j * 128) + 15) >> 4
        return lax.fori_loop(0, hi, ext, acc)

    def cond_k(k):
        t = k >> 2
        j = k & 3
        return site_nt(t) > j * 128

    nsites = (NBUCKET - 1) * NCHUNK

    def pair(i, a):
        k0 = 2 * i
        k1 = 2 * i + 1
        c0 = cond_k(k0)
        c1 = cond_k(k1)
        jax.lax.cond(c0, lambda: fire(k0, 0), lambda: None)
        jax.lax.cond(c1, lambda: fire(k1, 1), lambda: None)
        a = jax.lax.cond(c0, lambda a: drain_extract(k0, 0, a),
                         lambda a: a, a)
        a = jax.lax.cond(c1, lambda a: drain_extract(k1, 1, a),
                         lambda a: a, a)
        return a

    a = lax.fori_loop(0, nsites // 2 + dyn, pair, jnp.zeros((16,), jnp.float32))

    def drain7(i, _):
        pltpu.make_async_copy(yp_hbm.at[0, pl.ds(896, 104)], b7buf.at[i],
                              sem7).wait()
        return 0

    lax.fori_loop(0, n7, drain7, 0)

    def ext7(h, a):
        lane16 = lanebuf[pl.ds(7 * PER_TILE + h * 16, 16)]
        slot16 = h * 16 + iota
        vals = plsc.load_gather(b7buf, [slot16, lane16])
        valid = slot16 < n7
        return a + jnp.where(valid, _log16(vals + 1e-7), 0.0)

    a = lax.fori_loop(0, (n7 + 15) >> 4, ext7, a)

    stage_v[...] = a
    pltpu.sync_copy(stage_v, out_hbm.at[wid])


def _tc_finish_body(part_ref, out_ref):
    out_ref[0, 0] = jnp.sum(part_ref[...]) * (-1.0 / B)


@jax.jit
def kernel(y_pred, y_true):
    yt = y_true.astype(jnp.int32)
    mesh = plsc.VectorSubcoreMesh(
        core_axis_name="c", subcore_axis_name="s", num_cores=2)
    cp = pltpu.CompilerParams()
    if "needs_layout_passes" in pltpu.CompilerParams.__dataclass_fields__:
        cp = dataclasses.replace(cp, needs_layout_passes=False)
    run = pl.kernel(
        _sc_body,
        out_type=jax.ShapeDtypeStruct((NTILE, 16), jnp.float32),
        mesh=mesh,
        scratch_types=[
            pltpu.VMEM((PER_TILE,), jnp.int32),
            pltpu.VMEM((NBUCKET * PER_TILE,), jnp.int32),
            pltpu.VMEM((NBUCKET * PER_TILE,), jnp.int32),
            pltpu.VMEM((128, 128), jnp.float32),
            pltpu.VMEM((128, 128), jnp.float32),
            pltpu.VMEM((PER_TILE, 104), jnp.float32),
            pltpu.VMEM((16,), jnp.float32),
            pltpu.SemaphoreType.DMA,
            pltpu.SemaphoreType.DMA,
            pltpu.SemaphoreType.DMA,
        ],
        compiler_params=cp,
    )
    part = run(y_pred, yt)
    loss = pl.pallas_call(
        _tc_finish_body,
        out_shape=jax.ShapeDtypeStruct((1, 1), jnp.float32),
        out_specs=pl.BlockSpec(memory_space=pltpu.SMEM),
    )(part)
    return loss[0, 0]
